# baseline (speedup 1.0000x reference)
import functools

import jax
import jax.numpy as jnp
from jax import lax
from jax.experimental import pallas as pl
from jax.experimental.pallas import tpu as pltpu
from jax.experimental.pallas import tpu_sc as plsc

B = 4096
NUM_NEG = 128
D = 128
GAMMA = 24.0

NC = 2
NS = 16
NW = NC * NS
BW = B // NW
NOUT = 1 + NUM_NEG
NGROUP = NUM_NEG // 16


def _body(q0_hbm, q1_hbm, pos_hbm, neg_hbm, ent_hbm, rel_hbm, pout_hbm, nout_hbm,
          idxq0_v, idxq1_v, idxpos_v, idxneg_v,
          center_v, rel_v, negbuf0, negbuf1, out_v, poslog_v, trans_v,
          semg, sem0, sem1):
    wid = lax.axis_index("s") * NC + lax.axis_index("c")
    base = wid * BW
    lane = lax.iota(jnp.int32, 16)

    pltpu.sync_copy(neg_hbm.at[pl.ds(base, BW)], idxneg_v)
    pltpu.async_copy(ent_hbm.at[idxneg_v.at[0]], negbuf0, sem0)
    pltpu.async_copy(ent_hbm.at[idxneg_v.at[1]], negbuf1, sem1)

    pltpu.sync_copy(q0_hbm.at[pl.ds(base, BW)], idxq0_v)
    pltpu.sync_copy(q1_hbm.at[pl.ds(base, BW)], idxq1_v)
    pltpu.sync_copy(pos_hbm.at[pl.ds(base, BW)], idxpos_v)
    cg = pltpu.async_copy(ent_hbm.at[idxq0_v], center_v, semg)
    rg = pltpu.async_copy(rel_hbm.at[idxq1_v], rel_v, semg)
    cg.wait()
    rg.wait()

    def center_body(r, carry):
        for c in range(D // 16):
            sl = pl.ds(c * 16, 16)
            center_v[r, sl] = center_v[r, sl] + rel_v[r, sl]
        return carry
    lax.fori_loop(0, BW, center_body, 0)

    posrows_v = rel_v
    pltpu.async_copy(ent_hbm.at[idxpos_v], posrows_v, semg).wait()

    colidx = [jnp.full((16,), k, dtype=jnp.int32) for k in range(16)]

    def l1_rows16(src_v, srow0, cen_rows):
        for j in range(16):
            r = srow0 + j
            if cen_rows is None:
                cvs = [center_v[r, pl.ds(c * 16, 16)] for c in range(8)]
            else:
                cvs = cen_rows
            d0 = [jnp.abs(src_v[r, pl.ds(c * 16, 16)] - cvs[c]) for c in range(8)]
            d1 = [d0[2 * c] + d0[2 * c + 1] for c in range(4)]
            d2 = [d1[0] + d1[1], d1[2] + d1[3]]
            trans_v[j, pl.ds(0, 16)] = d2[0] + d2[1]
        cols = [plsc.load_gather(trans_v, [lane, colidx[k]]) for k in range(16)]
        while len(cols) > 1:
            cols = [cols[2 * i] + cols[2 * i + 1] for i in range(len(cols) // 2)]
        return cols[0]

    def pos_body(jg, carry):
        tsum = l1_rows16(posrows_v, jg * 16, None)
        poslog_v[pl.ds(jg * 16, 16)] = GAMMA - tsum
        return carry
    lax.fori_loop(0, BW // 16, pos_body, 0)

    def neg_row(bb, buf):
        cen_rows = [center_v[bb, pl.ds(c * 16, 16)] for c in range(8)]

        def jg_body(jg, carry):
            tsum = l1_rows16(buf, jg * 16, cen_rows)
            out_v[bb, pl.ds(jg * 16, 16)] = GAMMA - tsum
            return carry
        lax.fori_loop(0, NUM_NEG // 16, jg_body, 0)

    def outer(i, carry):
        bo = i * 2
        for t, (buf, sem) in enumerate(((negbuf0, sem0), (negbuf1, sem1))):
            bb = bo + t
            pltpu.make_async_copy(ent_hbm.at[pl.ds(0, NUM_NEG)], buf, sem).wait()
            neg_row(bb, buf)
            nxt = jnp.minimum(bb + 2, BW - 1)
            pltpu.async_copy(ent_hbm.at[idxneg_v.at[nxt]], buf, sem)
        return carry
    lax.fori_loop(0, BW // 2, outer, 0)

    pltpu.make_async_copy(ent_hbm.at[pl.ds(0, NUM_NEG)], negbuf0, sem0).wait()
    pltpu.make_async_copy(ent_hbm.at[pl.ds(0, NUM_NEG)], negbuf1, sem1).wait()

    pltpu.sync_copy(out_v, nout_hbm.at[pl.ds(base, BW)])
    pltpu.sync_copy(poslog_v, pout_hbm.at[pl.ds(base, BW)])


_mesh = plsc.VectorSubcoreMesh(core_axis_name="c", subcore_axis_name="s",
                               num_cores=NC, num_subcores=NS)

_sc_call = functools.partial(
    pl.kernel,
    out_type=(jax.ShapeDtypeStruct((B,), jnp.float32),
              jax.ShapeDtypeStruct((B, NUM_NEG), jnp.float32)),
    mesh=_mesh,
    compiler_params=pltpu.CompilerParams(needs_layout_passes=False),
    scratch_types=[
        pltpu.VMEM((BW,), jnp.int32),
        pltpu.VMEM((BW,), jnp.int32),
        pltpu.VMEM((BW,), jnp.int32),
        pltpu.VMEM((BW, NUM_NEG), jnp.int32),
        pltpu.VMEM((BW, D), jnp.float32),
        pltpu.VMEM((BW, D), jnp.float32),
        pltpu.VMEM((NUM_NEG, D), jnp.float32),
        pltpu.VMEM((NUM_NEG, D), jnp.float32),
        pltpu.VMEM((BW, NUM_NEG), jnp.float32),
        pltpu.VMEM((BW,), jnp.float32),
        pltpu.VMEM((16, 17), jnp.float32),
        pltpu.SemaphoreType.DMA,
        pltpu.SemaphoreType.DMA,
        pltpu.SemaphoreType.DMA,
    ],
)(_body)


def kernel(positive_sample, negative_sample, subsampling_weight, queries,
           ent_embedding, rel_embedding):
    del subsampling_weight
    q0 = queries[:, 0].astype(jnp.int32)
    q1 = queries[:, 1].astype(jnp.int32)
    pos = positive_sample.astype(jnp.int32)
    neg = negative_sample.astype(jnp.int32)
    pos_logit, neg_logit = _sc_call(q0, q1, pos, neg, ent_embedding, rel_embedding)
    return jnp.concatenate([pos_logit[:, None], neg_logit], axis=1)


# speedup vs baseline: 1.5107x; 1.5107x over previous
import functools

import jax
import jax.numpy as jnp
from jax import lax
from jax.experimental import pallas as pl
from jax.experimental.pallas import tpu as pltpu
from jax.experimental.pallas import tpu_sc as plsc

B = 4096
NUM_NEG = 128
D = 128
GAMMA = 24.0

NC = 2
NS = 16
NW = NC * NS
BW = B // NW
NOUT = 1 + NUM_NEG
NGROUP = NUM_NEG // 16


def _body(q0_hbm, q1_hbm, pos_hbm, neg_hbm, ent_hbm, rel_hbm, pout_hbm, nout_hbm,
          idxq0_v, idxq1_v, idxpos_v, idxneg_v,
          center_v, rel_v, negbuf0, negbuf1, out_v, poslog_v, trans_v,
          semg, sem0, sem1):
    wid = lax.axis_index("s") * NC + lax.axis_index("c")
    base = wid * BW
    lane = lax.iota(jnp.int32, 16)

    pltpu.sync_copy(neg_hbm.at[pl.ds(base, BW)], idxneg_v)
    pltpu.async_copy(ent_hbm.at[idxneg_v.at[0]], negbuf0, sem0)
    pltpu.async_copy(ent_hbm.at[idxneg_v.at[1]], negbuf1, sem1)

    pltpu.sync_copy(q0_hbm.at[pl.ds(base, BW)], idxq0_v)
    pltpu.sync_copy(q1_hbm.at[pl.ds(base, BW)], idxq1_v)
    pltpu.sync_copy(pos_hbm.at[pl.ds(base, BW)], idxpos_v)
    cg = pltpu.async_copy(ent_hbm.at[idxq0_v], center_v, semg)
    rg = pltpu.async_copy(rel_hbm.at[idxq1_v], rel_v, semg)
    cg.wait()
    rg.wait()

    def center_body(r, carry):
        for c in range(D // 16):
            sl = pl.ds(c * 16, 16)
            center_v[r, sl] = center_v[r, sl] + rel_v[r, sl]
        return carry
    lax.fori_loop(0, BW, center_body, 0)

    posrows_v = rel_v
    pltpu.async_copy(ent_hbm.at[idxpos_v], posrows_v, semg).wait()

    colidx = [jnp.full((16,), k, dtype=jnp.int32) for k in range(16)]

    def l1_rows16(src_v, srow0, cen_rows):
        ss = []
        for j in range(16):
            r = srow0 + j
            if cen_rows is None:
                cvs = [center_v[r, pl.ds(c * 16, 16)] for c in range(8)]
            else:
                cvs = cen_rows
            d0 = [jnp.abs(src_v[r, pl.ds(c * 16, 16)] - cvs[c]) for c in range(8)]
            d1 = [d0[2 * c] + d0[2 * c + 1] for c in range(4)]
            d2 = [d1[0] + d1[1], d1[2] + d1[3]]
            ss.append(d2[0] + d2[1])
        for j in range(16):
            trans_v[j, pl.ds(0, 16)] = ss[j]
        cols = [plsc.load_gather(trans_v, [lane, colidx[k]]) for k in range(16)]
        while len(cols) > 1:
            cols = [cols[2 * i] + cols[2 * i + 1] for i in range(len(cols) // 2)]
        return cols[0]

    def pos_body(jg, carry):
        tsum = l1_rows16(posrows_v, jg * 16, None)
        poslog_v[pl.ds(jg * 16, 16)] = GAMMA - tsum
        return carry
    lax.fori_loop(0, BW // 16, pos_body, 0)

    def neg_row(bb, buf):
        cen_rows = [center_v[bb, pl.ds(c * 16, 16)] for c in range(8)]

        def jg_body(jg, carry):
            tsum = l1_rows16(buf, jg * 16, cen_rows)
            out_v[bb, pl.ds(jg * 16, 16)] = GAMMA - tsum
            return carry
        lax.fori_loop(0, NUM_NEG // 16, jg_body, 0)

    def outer(i, carry):
        bo = i * 2
        for t, (buf, sem) in enumerate(((negbuf0, sem0), (negbuf1, sem1))):
            bb = bo + t
            pltpu.make_async_copy(ent_hbm.at[pl.ds(0, NUM_NEG)], buf, sem).wait()
            neg_row(bb, buf)
            nxt = jnp.minimum(bb + 2, BW - 1)
            pltpu.async_copy(ent_hbm.at[idxneg_v.at[nxt]], buf, sem)
        return carry
    lax.fori_loop(0, BW // 2, outer, 0)

    pltpu.make_async_copy(ent_hbm.at[pl.ds(0, NUM_NEG)], negbuf0, sem0).wait()
    pltpu.make_async_copy(ent_hbm.at[pl.ds(0, NUM_NEG)], negbuf1, sem1).wait()

    pltpu.sync_copy(out_v, nout_hbm.at[pl.ds(base, BW)])
    pltpu.sync_copy(poslog_v, pout_hbm.at[pl.ds(base, BW)])


_mesh = plsc.VectorSubcoreMesh(core_axis_name="c", subcore_axis_name="s",
                               num_cores=NC, num_subcores=NS)

_sc_call = functools.partial(
    pl.kernel,
    out_type=(jax.ShapeDtypeStruct((B,), jnp.float32),
              jax.ShapeDtypeStruct((B, NUM_NEG), jnp.float32)),
    mesh=_mesh,
    compiler_params=pltpu.CompilerParams(needs_layout_passes=False),
    scratch_types=[
        pltpu.VMEM((BW,), jnp.int32),
        pltpu.VMEM((BW,), jnp.int32),
        pltpu.VMEM((BW,), jnp.int32),
        pltpu.VMEM((BW, NUM_NEG), jnp.int32),
        pltpu.VMEM((BW, D), jnp.float32),
        pltpu.VMEM((BW, D), jnp.float32),
        pltpu.VMEM((NUM_NEG, D), jnp.float32),
        pltpu.VMEM((NUM_NEG, D), jnp.float32),
        pltpu.VMEM((BW, NUM_NEG), jnp.float32),
        pltpu.VMEM((BW,), jnp.float32),
        pltpu.VMEM((16, 17), jnp.float32),
        pltpu.SemaphoreType.DMA,
        pltpu.SemaphoreType.DMA,
        pltpu.SemaphoreType.DMA,
    ],
)(_body)


def kernel(positive_sample, negative_sample, subsampling_weight, queries,
           ent_embedding, rel_embedding):
    del subsampling_weight
    q0 = queries[:, 0].astype(jnp.int32)
    q1 = queries[:, 1].astype(jnp.int32)
    pos = positive_sample.astype(jnp.int32)
    neg = negative_sample.astype(jnp.int32)
    pos_logit, neg_logit = _sc_call(q0, q1, pos, neg, ent_embedding, rel_embedding)
    return jnp.concatenate([pos_logit[:, None], neg_logit], axis=1)
